# baseline (speedup 1.0000x reference)
import functools

import jax
import jax.numpy as jnp
from jax import lax
from jax.experimental import pallas as pl
from jax.experimental.pallas import tpu as pltpu
from jax.experimental.pallas import tpu_sc as plsc

_NUM_CLASSES = 8
_NEG_POS_RATIO = 3

_B, _H, _W, _C = 4, 512, 512, _NUM_CLASSES
_N = _H * _W * _C
_NCORE = 2
_NSUB = 16
_BPC = _B // _NCORE
_NSL = _N // _NSUB
_CH = 8192
_NCHUNK = _NSL // _CH
_VPC = _CH // 16
_POS_ROW = 128
_MSE_ROW = 129


def _sc_body(y_hbm, o_hbm, w_hbm, ts_hbm, out_hbm, neg_hbm,
             ybuf, obuf, wbuf, negbuf, cnt_h, sum_h, gcnt, gsum,
             tsbuf, outbuf, idx_lo, idx_hi, sh_cnt, sh_sum):
    core = lax.axis_index("c")
    sid = lax.axis_index("s")
    lanes = lax.iota(jnp.int32, 16)
    onesf = jnp.full((16,), 1.0, jnp.float32)
    zerof = jnp.zeros((16,), jnp.float32)

    def init_idx(i, _):
        v = lanes + i * 16
        idx_lo[pl.ds(i * 16, 16)] = v
        idx_hi[pl.ds(i * 16, 16)] = v + 128
        return 0

    lax.fori_loop(0, 8, init_idx, 0)
    pltpu.sync_copy(ts_hbm, tsbuf)

    def zero_local(i, _):
        cnt_h[i] = zerof
        sum_h[i] = zerof
        return 0

    def reset_hists():
        plsc.subcore_barrier()
        lax.fori_loop(0, 256, zero_local, 0)

        @pl.when(sid == 0)
        def _():
            pltpu.sync_copy(cnt_h, sh_cnt)
            pltpu.sync_copy(sum_h, sh_sum)

        plsc.subcore_barrier()

    def combine_hists():
        pltpu.sync_copy(cnt_h.at[pl.ds(0, 128)], sh_cnt.at[idx_lo], add=True)
        pltpu.sync_copy(cnt_h.at[pl.ds(128, 128)], sh_cnt.at[idx_hi], add=True)
        pltpu.sync_copy(sum_h.at[pl.ds(0, 128)], sh_sum.at[idx_lo], add=True)
        pltpu.sync_copy(sum_h.at[pl.ds(128, 128)], sh_sum.at[idx_hi], add=True)
        plsc.subcore_barrier()
        pltpu.sync_copy(sh_cnt, gcnt)
        pltpu.sync_copy(sh_sum, gsum)

    def search(kk, top_bucket):
        def body(j, carry):
            cum, above, t, r, cnt_t, sum_t, found = carry
            b = top_bucket - 1 - j
            cj = jnp.sum(gcnt[b])
            sj = jnp.sum(gsum[b])
            here = jnp.logical_and(jnp.logical_not(found), cum + cj >= kk)
            t = jnp.where(here, b, t)
            r = jnp.where(here, kk - cum, r)
            cnt_t = jnp.where(here, cj, cnt_t)
            sum_t = jnp.where(here, sj, sum_t)
            above = jnp.where(jnp.logical_or(found, here), above, above + sj)
            return (cum + cj, above, t, r, cnt_t, sum_t,
                    jnp.logical_or(found, here))

        init = (jnp.float32(0), jnp.float32(0), jnp.int32(-1), jnp.float32(0),
                jnp.float32(0), jnp.float32(0), False)
        _, above, t, r, cnt_t, sum_t, _ = lax.fori_loop(
            0, top_bucket, body, init)
        return above, t, r, cnt_t, sum_t

    def sweep1(base):
        def chunk_body(ci, carry):
            pos_a, mse_a = carry
            off = base + ci * _CH
            pltpu.sync_copy(y_hbm.at[pl.ds(off, _CH)], ybuf)
            pltpu.sync_copy(o_hbm.at[pl.ds(off, _CH)], obuf)
            pltpu.sync_copy(w_hbm.at[pl.ds(off, _CH)], wbuf)

            def vbody(vi, c2):
                pa, ma = c2
                sl = pl.ds(vi * 16, 16)
                yv = ybuf[sl]
                ov = obuf[sl]
                wv = wbuf[sl]
                d = ov - yv
                m = d * d
                ma = ma + m
                posm = wv > 0
                pa = pa + jnp.where(posm, wv * m, 0.0)
                negm = jnp.logical_and(ov > 0, jnp.logical_not(posm))
                nv = jnp.where(negm, m, 0.0)
                negbuf[sl] = nv
                bits = plsc.bitcast(nv, jnp.int32)
                b1 = bits >> 24
                plsc.addupdate_scatter(cnt_h, [b1, lanes], onesf)
                plsc.addupdate_scatter(sum_h, [b1, lanes], nv)
                return (pa, ma)

            pos_a, mse_a = lax.fori_loop(0, _VPC, vbody, (pos_a, mse_a))
            pltpu.sync_copy(negbuf, neg_hbm.at[pl.ds(off, _CH)])
            return (pos_a, mse_a)

        pos_a, mse_a = lax.fori_loop(0, _NCHUNK, chunk_body, (zerof, zerof))
        cnt_h[_POS_ROW] = pos_a
        cnt_h[_MSE_ROW] = mse_a

    def sweep_refine(base, shift, prev_shift, prefix):
        def chunk_body(ci, _):
            off = base + ci * _CH
            pltpu.sync_copy(neg_hbm.at[pl.ds(off, _CH)], negbuf)

            def vbody(vi, __):
                sl = pl.ds(vi * 16, 16)
                nv = negbuf[sl]
                bits = plsc.bitcast(nv, jnp.int32)
                sel = (bits >> prev_shift) == prefix
                bb = (bits >> shift) & 0xFF
                plsc.addupdate_scatter(cnt_h, [bb, lanes], onesf, mask=sel)
                plsc.addupdate_scatter(sum_h, [bb, lanes], nv, mask=sel)
                return 0

            lax.fori_loop(0, _VPC, vbody, 0)
            return 0

        lax.fori_loop(0, _NCHUNK, chunk_body, 0)

    kvec = jnp.minimum(tsbuf[...].astype(jnp.int32) * _NEG_POS_RATIO,
                       _N).astype(jnp.float32)
    loss_acc = zerof
    mse_acc = zerof
    for bi in range(_BPC):
        b = core * _BPC + bi
        base = b * _N + sid * _NSL
        bsel = lanes == b
        kk = jnp.sum(jnp.where(bsel, kvec, 0.0))
        tsb = jnp.sum(jnp.where(bsel, tsbuf[...], 0.0))

        with jax.named_scope("reset1"):
            reset_hists()
        with jax.named_scope("sweep1"):
            sweep1(base)
        with jax.named_scope("combine1"):
            combine_hists()
        with jax.named_scope("search1"):
            above1, t1, r1, _, _ = search(kk, 128)
        pos_b = jnp.sum(gcnt[_POS_ROW])
        mse_b = jnp.sum(gcnt[_MSE_ROW])

        with jax.named_scope("reset2"):
            reset_hists()
        with jax.named_scope("sweep2"):
            sweep_refine(base, 16, 24, t1)
        with jax.named_scope("combine2"):
            combine_hists()
        with jax.named_scope("search2"):
            above2, t2, r2, _, _ = search(r1, 256)

        with jax.named_scope("reset3"):
            reset_hists()
        with jax.named_scope("sweep3"):
            sweep_refine(base, 8, 16, (t1 << 8) | t2)
        with jax.named_scope("combine3"):
            combine_hists()
        with jax.named_scope("search3"):
            above3, _, r3, cnt3, sum3 = search(r2, 256)

        mean3_v = jnp.full((16,), sum3) / jnp.maximum(jnp.full((16,), cnt3),
                                                      1.0)
        num_v = jnp.full((16,), pos_b + above1 + above2 + above3) \
            + r3 * mean3_v
        ts_v = jnp.full((16,), tsb)
        safe_ts = jnp.where(ts_v > 0, ts_v, 1.0)
        loss_acc = loss_acc + jnp.where(ts_v > 0, num_v / safe_ts, 0.0)
        mse_acc = mse_acc + mse_b

    plsc.subcore_barrier()

    @pl.when(sid == 0)
    def _():
        outv = jnp.where(lanes == 0, loss_acc,
                         jnp.where(lanes == 1, mse_acc, 0.0))
        outbuf[...] = outv
        pltpu.sync_copy(outbuf, out_hbm.at[core])


@functools.partial(jax.jit, static_argnames=())
def kernel(y, out, w, total_size):
    assert y.shape == (_B, _H, _W, _C)
    y2 = y.reshape(-1)
    o2 = out.reshape(-1)
    w2 = w.reshape(-1)
    ts_pad = jnp.zeros((16,), jnp.float32).at[:_B].set(
        total_size.reshape(-1).astype(jnp.float32))

    mesh = plsc.VectorSubcoreMesh(core_axis_name="c", subcore_axis_name="s",
                                  num_cores=_NCORE, num_subcores=_NSUB)
    fn = pl.kernel(
        _sc_body,
        out_type=(
            jax.ShapeDtypeStruct((_NCORE, 16), jnp.float32),
            jax.ShapeDtypeStruct((_B * _N,), jnp.float32),
        ),
        mesh=mesh,
        compiler_params=pltpu.CompilerParams(needs_layout_passes=False, use_tc_tiling_on_sc=False),
        scratch_types=[
            pltpu.VMEM((_CH,), jnp.float32),
            pltpu.VMEM((_CH,), jnp.float32),
            pltpu.VMEM((_CH,), jnp.float32),
            pltpu.VMEM((_CH,), jnp.float32),
            pltpu.VMEM((256, 16), jnp.float32),
            pltpu.VMEM((256, 16), jnp.float32),
            pltpu.VMEM((256, 16), jnp.float32),
            pltpu.VMEM((256, 16), jnp.float32),
            pltpu.VMEM((16,), jnp.float32),
            pltpu.VMEM((16,), jnp.float32),
            pltpu.VMEM((128,), jnp.int32),
            pltpu.VMEM((128,), jnp.int32),
            pltpu.VMEM_SHARED((256, 16), jnp.float32),
            pltpu.VMEM_SHARED((256, 16), jnp.float32),
        ],
    )
    partials, _neg = fn(y2, o2, w2, ts_pad)
    train_loss = (partials[0, 0] + partials[1, 0]) / _B
    mse_mean = (partials[0, 1] + partials[1, 1]) / (_B * _N)
    return ((train_loss + mse_mean) * 10).reshape(())


# speedup vs baseline: 2.7204x; 2.7204x over previous
import functools

import jax
import jax.numpy as jnp
from jax import lax
from jax.experimental import pallas as pl
from jax.experimental.pallas import tpu as pltpu
from jax.experimental.pallas import tpu_sc as plsc

_NUM_CLASSES = 8
_NEG_POS_RATIO = 3

_B, _H, _W, _C = 4, 512, 512, _NUM_CLASSES
_N = _H * _W * _C
_NCORE = 2
_NSUB = 16
_BPC = _B // _NCORE
_NSL = _N // _NSUB
_CH = 8192
_NCHUNK = _NSL // _CH
_VPC = _CH // 16
_POS_ROW = 128
_MSE_ROW = 129


def _sc_body(y_hbm, o_hbm, w_hbm, ts_hbm, out_hbm, neg_hbm,
             ybuf, obuf, wbuf, negbuf, cnt_h, sum_h, gcnt, gsum,
             tsbuf, outbuf, idx_lo, idx_hi, sh_cnt, sh_sum):
    core = lax.axis_index("c")
    sid = lax.axis_index("s")
    lanes = lax.iota(jnp.int32, 16)
    onesf = jnp.full((16,), 1.0, jnp.float32)
    zerof = jnp.zeros((16,), jnp.float32)

    def init_idx(i, _):
        v = lanes + i * 16
        idx_lo[pl.ds(i * 16, 16)] = v
        idx_hi[pl.ds(i * 16, 16)] = v + 128
        return 0

    lax.fori_loop(0, 8, init_idx, 0)
    pltpu.sync_copy(ts_hbm, tsbuf)

    def zero_local(i, _):
        cnt_h[i] = zerof
        sum_h[i] = zerof
        return 0

    def reset_hists():
        plsc.subcore_barrier()
        lax.fori_loop(0, 256, zero_local, 0)

        @pl.when(sid == 0)
        def _():
            pltpu.sync_copy(cnt_h, sh_cnt)
            pltpu.sync_copy(sum_h, sh_sum)

        plsc.subcore_barrier()

    def combine_hists():
        pltpu.sync_copy(cnt_h.at[pl.ds(0, 128)], sh_cnt.at[idx_lo], add=True)
        pltpu.sync_copy(cnt_h.at[pl.ds(128, 128)], sh_cnt.at[idx_hi], add=True)
        pltpu.sync_copy(sum_h.at[pl.ds(0, 128)], sh_sum.at[idx_lo], add=True)
        pltpu.sync_copy(sum_h.at[pl.ds(128, 128)], sh_sum.at[idx_hi], add=True)
        plsc.subcore_barrier()
        pltpu.sync_copy(sh_cnt, gcnt)
        pltpu.sync_copy(sh_sum, gsum)

    def search(kk, top_bucket):
        def body(j, carry):
            cum, above, t, r, cnt_t, sum_t, found = carry
            b = top_bucket - 1 - j
            cj = jnp.sum(gcnt[b])
            sj = jnp.sum(gsum[b])
            here = jnp.logical_and(jnp.logical_not(found), cum + cj >= kk)
            t = jnp.where(here, b, t)
            r = jnp.where(here, kk - cum, r)
            cnt_t = jnp.where(here, cj, cnt_t)
            sum_t = jnp.where(here, sj, sum_t)
            above = jnp.where(jnp.logical_or(found, here), above, above + sj)
            return (cum + cj, above, t, r, cnt_t, sum_t,
                    jnp.logical_or(found, here))

        init = (jnp.float32(0), jnp.float32(0), jnp.int32(-1), jnp.float32(0),
                jnp.float32(0), jnp.float32(0), False)
        _, above, t, r, cnt_t, sum_t, _ = lax.fori_loop(
            0, top_bucket, body, init)
        return above, t, r, cnt_t, sum_t

    def sweep1(base):
        def chunk_body(ci, carry):
            pos_a, mse_a = carry
            off = base + ci * _CH
            pltpu.sync_copy(y_hbm.at[pl.ds(off, _CH)], ybuf)
            pltpu.sync_copy(o_hbm.at[pl.ds(off, _CH)], obuf)
            pltpu.sync_copy(w_hbm.at[pl.ds(off, _CH)], wbuf)

            def vbody(vi, c2):
                pa, ma = c2
                sl = pl.ds(vi * 16, 16)
                yv = ybuf[sl]
                ov = obuf[sl]
                wv = wbuf[sl]
                d = ov - yv
                m = d * d
                ma = ma + m
                posm = wv > 0
                pa = pa + jnp.where(posm, wv * m, 0.0)
                negm = jnp.logical_and(ov > 0, jnp.logical_not(posm))
                nv = jnp.where(negm, m, 0.0)
                negbuf[sl] = nv
                bits = plsc.bitcast(nv, jnp.int32)
                b1 = bits >> 24
                plsc.addupdate_scatter(cnt_h, [b1, lanes], onesf)
                plsc.addupdate_scatter(sum_h, [b1, lanes], nv)
                return (pa, ma)

            pos_a, mse_a = lax.fori_loop(0, _VPC, vbody, (pos_a, mse_a))
            pltpu.sync_copy(negbuf, neg_hbm.at[pl.ds(off, _CH)])
            return (pos_a, mse_a)

        pos_a, mse_a = lax.fori_loop(0, _NCHUNK, chunk_body, (zerof, zerof))
        cnt_h[_POS_ROW] = pos_a
        cnt_h[_MSE_ROW] = mse_a

    def sweep_refine(base, shift, prev_shift, prefix):
        def chunk_body(ci, _):
            off = base + ci * _CH
            pltpu.sync_copy(neg_hbm.at[pl.ds(off, _CH)], negbuf)

            def vbody(vi, __):
                sl = pl.ds(vi * 16, 16)
                nv = negbuf[sl]
                bits = plsc.bitcast(nv, jnp.int32)
                sel = (bits >> prev_shift) == prefix
                bb = (bits >> shift) & 0xFF
                plsc.addupdate_scatter(cnt_h, [bb, lanes], onesf, mask=sel)
                plsc.addupdate_scatter(sum_h, [bb, lanes], nv, mask=sel)
                return 0

            lax.fori_loop(0, _VPC, vbody, 0)
            return 0

        lax.fori_loop(0, _NCHUNK, chunk_body, 0)

    kvec = jnp.minimum(tsbuf[...].astype(jnp.int32) * _NEG_POS_RATIO,
                       _N).astype(jnp.float32)
    loss_acc = zerof
    mse_acc = zerof
    for bi in range(_BPC):
        b = core * _BPC + bi
        base = b * _N + sid * _NSL
        bsel = lanes == b
        kk = jnp.sum(jnp.where(bsel, kvec, 0.0))
        tsb = jnp.sum(jnp.where(bsel, tsbuf[...], 0.0))

        with jax.named_scope("reset1"):
            reset_hists()
        with jax.named_scope("sweep1"):
            sweep1(base)
        with jax.named_scope("combine1"):
            combine_hists()
        with jax.named_scope("search1"):
            above1, t1, r1, _, _ = search(kk, 128)
        pos_b = jnp.sum(gcnt[_POS_ROW])
        mse_b = jnp.sum(gcnt[_MSE_ROW])

        with jax.named_scope("reset2"):
            reset_hists()
        with jax.named_scope("sweep2"):
            sweep_refine(base, 16, 24, t1)
        with jax.named_scope("combine2"):
            combine_hists()
        with jax.named_scope("search2"):
            above2, t2, r2, _, _ = search(r1, 256)

        with jax.named_scope("reset3"):
            reset_hists()
        with jax.named_scope("sweep3"):
            sweep_refine(base, 8, 16, (t1 << 8) | t2)
        with jax.named_scope("combine3"):
            combine_hists()
        with jax.named_scope("search3"):
            above3, _, r3, cnt3, sum3 = search(r2, 256)

        mean3_v = jnp.full((16,), sum3) / jnp.maximum(jnp.full((16,), cnt3),
                                                      1.0)
        num_v = jnp.full((16,), pos_b + above1 + above2 + above3) \
            + r3 * mean3_v
        ts_v = jnp.full((16,), tsb)
        safe_ts = jnp.where(ts_v > 0, ts_v, 1.0)
        loss_acc = loss_acc + jnp.where(ts_v > 0, num_v / safe_ts, 0.0)
        mse_acc = mse_acc + mse_b

    plsc.subcore_barrier()

    @pl.when(sid == 0)
    def _():
        outv = jnp.where(lanes == 0, loss_acc,
                         jnp.where(lanes == 1, mse_acc, 0.0))
        outbuf[...] = outv
        pltpu.sync_copy(outbuf, out_hbm.at[core])


@functools.partial(jax.jit, static_argnames=())
def kernel(y, out, w, total_size):
    assert y.shape == (_B, _H, _W, _C)

    def _flat_physical(a):
        return a.reshape(_B, _H, _W // 128, 128, _C).transpose(
            0, 1, 2, 4, 3).reshape(-1)

    y2 = _flat_physical(y)
    o2 = _flat_physical(out)
    w2 = _flat_physical(w)
    ts_pad = jnp.zeros((16,), jnp.float32).at[:_B].set(
        total_size.reshape(-1).astype(jnp.float32))

    mesh = plsc.VectorSubcoreMesh(core_axis_name="c", subcore_axis_name="s",
                                  num_cores=_NCORE, num_subcores=_NSUB)
    fn = pl.kernel(
        _sc_body,
        out_type=(
            jax.ShapeDtypeStruct((_NCORE, 16), jnp.float32),
            jax.ShapeDtypeStruct((_B * _N,), jnp.float32),
        ),
        mesh=mesh,
        compiler_params=pltpu.CompilerParams(needs_layout_passes=False, use_tc_tiling_on_sc=False),
        scratch_types=[
            pltpu.VMEM((_CH,), jnp.float32),
            pltpu.VMEM((_CH,), jnp.float32),
            pltpu.VMEM((_CH,), jnp.float32),
            pltpu.VMEM((_CH,), jnp.float32),
            pltpu.VMEM((256, 16), jnp.float32),
            pltpu.VMEM((256, 16), jnp.float32),
            pltpu.VMEM((256, 16), jnp.float32),
            pltpu.VMEM((256, 16), jnp.float32),
            pltpu.VMEM((16,), jnp.float32),
            pltpu.VMEM((16,), jnp.float32),
            pltpu.VMEM((128,), jnp.int32),
            pltpu.VMEM((128,), jnp.int32),
            pltpu.VMEM_SHARED((256, 16), jnp.float32),
            pltpu.VMEM_SHARED((256, 16), jnp.float32),
        ],
    )
    partials, _neg = fn(y2, o2, w2, ts_pad)
    train_loss = (partials[0, 0] + partials[1, 0]) / _B
    mse_mean = (partials[0, 1] + partials[1, 1]) / (_B * _N)
    return ((train_loss + mse_mean) * 10).reshape(())


# speedup vs baseline: 3.5101x; 1.2903x over previous
import functools

import jax
import jax.numpy as jnp
from jax import lax
from jax.experimental import pallas as pl
from jax.experimental.pallas import tpu as pltpu
from jax.experimental.pallas import tpu_sc as plsc

_NUM_CLASSES = 8
_NEG_POS_RATIO = 3

_B, _H, _W, _C = 4, 512, 512, _NUM_CLASSES
_N = _H * _W * _C
_NCORE = 2
_NSUB = 16
_BPC = _B // _NCORE
_NSL = _N // _NSUB
_CH = 8192
_NCHUNK = _NSL // _CH
_VPC = _CH // 16
_UNROLL = 4
_POS_ROW = 128
_MSE_ROW = 129


def _sc_body(y_hbm, o_hbm, w_hbm, ts_hbm, out_hbm, neg_hbm,
             ybuf, obuf, wbuf, negbuf, ybuf2, obuf2, wbuf2, negbuf2,
             cnt_h, sum_h, gcnt, gsum, tsbuf, outbuf, idx_lo, idx_hi,
             sem_a, sem_b, sem_na, sem_nb, sh_cnt, sh_sum):
    core = lax.axis_index("c")
    sid = lax.axis_index("s")
    lanes = lax.iota(jnp.int32, 16)
    onesf = jnp.full((16,), 1.0, jnp.float32)
    zerof = jnp.zeros((16,), jnp.float32)

    def init_idx(i, _):
        v = lanes + i * 16
        idx_lo[pl.ds(i * 16, 16)] = v
        idx_hi[pl.ds(i * 16, 16)] = v + 128
        return 0

    lax.fori_loop(0, 8, init_idx, 0)
    pltpu.sync_copy(ts_hbm, tsbuf)

    def zero_local(i, _):
        cnt_h[i] = zerof
        sum_h[i] = zerof
        return 0

    def reset_hists():
        plsc.subcore_barrier()
        lax.fori_loop(0, 256, zero_local, 0)

        @pl.when(sid == 0)
        def _():
            pltpu.sync_copy(cnt_h, sh_cnt)
            pltpu.sync_copy(sum_h, sh_sum)

        plsc.subcore_barrier()

    def combine_hists():
        pltpu.sync_copy(cnt_h.at[pl.ds(0, 128)], sh_cnt.at[idx_lo], add=True)
        pltpu.sync_copy(cnt_h.at[pl.ds(128, 128)], sh_cnt.at[idx_hi], add=True)
        pltpu.sync_copy(sum_h.at[pl.ds(0, 128)], sh_sum.at[idx_lo], add=True)
        pltpu.sync_copy(sum_h.at[pl.ds(128, 128)], sh_sum.at[idx_hi], add=True)
        plsc.subcore_barrier()
        pltpu.sync_copy(sh_cnt, gcnt)
        pltpu.sync_copy(sh_sum, gsum)

    def search(kk, top_bucket):
        def body(j, carry):
            cum, above, t, r, cnt_t, sum_t, found = carry
            b = top_bucket - 1 - j
            cj = jnp.sum(gcnt[b])
            sj = jnp.sum(gsum[b])
            here = jnp.logical_and(jnp.logical_not(found), cum + cj >= kk)
            t = jnp.where(here, b, t)
            r = jnp.where(here, kk - cum, r)
            cnt_t = jnp.where(here, cj, cnt_t)
            sum_t = jnp.where(here, sj, sum_t)
            above = jnp.where(jnp.logical_or(found, here), above, above + sj)
            return (cum + cj, above, t, r, cnt_t, sum_t,
                    jnp.logical_or(found, here))

        init = (jnp.float32(0), jnp.float32(0), jnp.int32(-1), jnp.float32(0),
                jnp.float32(0), jnp.float32(0), False)
        _, above, t, r, cnt_t, sum_t, _ = lax.fori_loop(
            0, top_bucket, body, init)
        return above, t, r, cnt_t, sum_t

    def start_loads(off, yb, ob, wb, sem):
        pltpu.async_copy(y_hbm.at[pl.ds(off, _CH)], yb, sem)
        pltpu.async_copy(o_hbm.at[pl.ds(off, _CH)], ob, sem)
        pltpu.async_copy(w_hbm.at[pl.ds(off, _CH)], wb, sem)

    def wait_loads(yb, ob, wb, sem):
        pltpu.make_async_copy(y_hbm.at[pl.ds(0, _CH)], yb, sem).wait()
        pltpu.make_async_copy(o_hbm.at[pl.ds(0, _CH)], ob, sem).wait()
        pltpu.make_async_copy(w_hbm.at[pl.ds(0, _CH)], wb, sem).wait()

    def start_neg_load(off, nb, sem):
        pltpu.async_copy(neg_hbm.at[pl.ds(off, _CH)], nb, sem)

    def wait_neg_load(nb, sem):
        pltpu.make_async_copy(neg_hbm.at[pl.ds(0, _CH)], nb, sem).wait()

    def start_neg_store(nb, off, sem):
        pltpu.async_copy(nb, neg_hbm.at[pl.ds(off, _CH)], sem)

    def wait_neg_store(nb, sem):
        pltpu.make_async_copy(nb, neg_hbm.at[pl.ds(0, _CH)], sem).wait()

    def compute_chunk(yb, ob, wb, nb, carry):
        def vbody(vi, c2):
            pa, ma = c2
            for u in range(_UNROLL):
                sl = pl.ds((vi * _UNROLL + u) * 16, 16)
                yv = yb[sl]
                ov = ob[sl]
                wv = wb[sl]
                d = ov - yv
                m = d * d
                ma = ma + m
                posm = wv > 0
                pa = pa + jnp.where(posm, wv * m, 0.0)
                negm = jnp.logical_and(ov > 0, jnp.logical_not(posm))
                nv = jnp.where(negm, m, 0.0)
                nb[sl] = nv
                bits = plsc.bitcast(nv, jnp.int32)
                b1 = bits >> 24
                plsc.addupdate_scatter(cnt_h, [b1, lanes], onesf)
                plsc.addupdate_scatter(sum_h, [b1, lanes], nv)
            return (pa, ma)

        return lax.fori_loop(0, _VPC // _UNROLL, vbody, carry)

    def sweep1(base):
        start_loads(base, ybuf, obuf, wbuf, sem_a)

        def pair(i, carry):
            off_a = base + (2 * i) * _CH
            wait_loads(ybuf, obuf, wbuf, sem_a)
            start_loads(off_a + _CH, ybuf2, obuf2, wbuf2, sem_b)

            @pl.when(i > 0)
            def _():
                wait_neg_store(negbuf, sem_na)

            carry = compute_chunk(ybuf, obuf, wbuf, negbuf, carry)
            start_neg_store(negbuf, off_a, sem_na)

            wait_loads(ybuf2, obuf2, wbuf2, sem_b)

            @pl.when(i < _NCHUNK // 2 - 1)
            def _():
                start_loads(off_a + 2 * _CH, ybuf, obuf, wbuf, sem_a)

            @pl.when(i > 0)
            def _():
                wait_neg_store(negbuf2, sem_nb)

            carry = compute_chunk(ybuf2, obuf2, wbuf2, negbuf2, carry)
            start_neg_store(negbuf2, off_a + _CH, sem_nb)
            return carry

        pos_a, mse_a = lax.fori_loop(0, _NCHUNK // 2, pair, (zerof, zerof))
        wait_neg_store(negbuf, sem_na)
        wait_neg_store(negbuf2, sem_nb)
        cnt_h[_POS_ROW] = pos_a
        cnt_h[_MSE_ROW] = mse_a

    def hist_chunk(nb, shift, prev_shift, prefix):
        def vbody(vi, _):
            for u in range(_UNROLL):
                sl = pl.ds((vi * _UNROLL + u) * 16, 16)
                nv = nb[sl]
                bits = plsc.bitcast(nv, jnp.int32)
                sel = (bits >> prev_shift) == prefix
                bb = (bits >> shift) & 0xFF
                plsc.addupdate_scatter(cnt_h, [bb, lanes], onesf, mask=sel)
                plsc.addupdate_scatter(sum_h, [bb, lanes], nv, mask=sel)
            return 0

        lax.fori_loop(0, _VPC // _UNROLL, vbody, 0)

    def sweep_refine(base, shift, prev_shift, prefix):
        start_neg_load(base, ybuf, sem_a)

        def pair(i, _):
            off_a = base + (2 * i) * _CH
            wait_neg_load(ybuf, sem_a)
            start_neg_load(off_a + _CH, ybuf2, sem_b)
            hist_chunk(ybuf, shift, prev_shift, prefix)
            wait_neg_load(ybuf2, sem_b)

            @pl.when(i < _NCHUNK // 2 - 1)
            def _():
                start_neg_load(off_a + 2 * _CH, ybuf, sem_a)

            hist_chunk(ybuf2, shift, prev_shift, prefix)
            return 0

        lax.fori_loop(0, _NCHUNK // 2, pair, 0)

    kvec = jnp.minimum(tsbuf[...].astype(jnp.int32) * _NEG_POS_RATIO,
                       _N).astype(jnp.float32)

    def batch_body(bi, acc):
        loss_acc, mse_acc = acc
        b = core * _BPC + bi
        base = b * _N + sid * _NSL
        bsel = lanes == b
        kk = jnp.sum(jnp.where(bsel, kvec, 0.0))
        tsb = jnp.sum(jnp.where(bsel, tsbuf[...], 0.0))

        with jax.named_scope("reset1"):
            reset_hists()
        with jax.named_scope("sweep1"):
            sweep1(base)
        with jax.named_scope("combine1"):
            combine_hists()
        with jax.named_scope("search1"):
            above1, t1, r1, _, _ = search(kk, 128)
        pos_b = jnp.sum(gcnt[_POS_ROW])
        mse_b = jnp.sum(gcnt[_MSE_ROW])

        with jax.named_scope("reset2"):
            reset_hists()
        with jax.named_scope("sweep2"):
            sweep_refine(base, 16, 24, t1)
        with jax.named_scope("combine2"):
            combine_hists()
        with jax.named_scope("search2"):
            above2, t2, r2, _, _ = search(r1, 256)

        with jax.named_scope("reset3"):
            reset_hists()
        with jax.named_scope("sweep3"):
            sweep_refine(base, 8, 16, (t1 << 8) | t2)
        with jax.named_scope("combine3"):
            combine_hists()
        with jax.named_scope("search3"):
            above3, _, r3, cnt3, sum3 = search(r2, 256)

        mean3_v = jnp.full((16,), sum3) / jnp.maximum(jnp.full((16,), cnt3),
                                                      1.0)
        num_v = jnp.full((16,), pos_b + above1 + above2 + above3) \
            + r3 * mean3_v
        ts_v = jnp.full((16,), tsb)
        safe_ts = jnp.where(ts_v > 0, ts_v, 1.0)
        loss_acc = loss_acc + jnp.where(ts_v > 0, num_v / safe_ts, 0.0)
        mse_acc = mse_acc + mse_b
        return (loss_acc, mse_acc)

    loss_acc, mse_acc = lax.fori_loop(0, _BPC, batch_body, (zerof, zerof))

    plsc.subcore_barrier()

    @pl.when(sid == 0)
    def _():
        outv = jnp.where(lanes == 0, loss_acc,
                         jnp.where(lanes == 1, mse_acc, 0.0))
        outbuf[...] = outv
        pltpu.sync_copy(outbuf, out_hbm.at[core])


@functools.partial(jax.jit, static_argnames=())
def kernel(y, out, w, total_size):
    assert y.shape == (_B, _H, _W, _C)

    def _flat_physical(a):
        return a.reshape(_B, _H, _W // 128, 128, _C).transpose(
            0, 1, 2, 4, 3).reshape(-1)

    y2 = _flat_physical(y)
    o2 = _flat_physical(out)
    w2 = _flat_physical(w)
    ts_pad = jnp.zeros((16,), jnp.float32).at[:_B].set(
        total_size.reshape(-1).astype(jnp.float32))

    mesh = plsc.VectorSubcoreMesh(core_axis_name="c", subcore_axis_name="s",
                                  num_cores=_NCORE, num_subcores=_NSUB)
    fn = pl.kernel(
        _sc_body,
        out_type=(
            jax.ShapeDtypeStruct((_NCORE, 16), jnp.float32),
            jax.ShapeDtypeStruct((_B * _N,), jnp.float32),
        ),
        mesh=mesh,
        compiler_params=pltpu.CompilerParams(needs_layout_passes=False,
                                             use_tc_tiling_on_sc=False),
        scratch_types=[
            pltpu.VMEM((_CH,), jnp.float32),
            pltpu.VMEM((_CH,), jnp.float32),
            pltpu.VMEM((_CH,), jnp.float32),
            pltpu.VMEM((_CH,), jnp.float32),
            pltpu.VMEM((_CH,), jnp.float32),
            pltpu.VMEM((_CH,), jnp.float32),
            pltpu.VMEM((_CH,), jnp.float32),
            pltpu.VMEM((_CH,), jnp.float32),
            pltpu.VMEM((256, 16), jnp.float32),
            pltpu.VMEM((256, 16), jnp.float32),
            pltpu.VMEM((256, 16), jnp.float32),
            pltpu.VMEM((256, 16), jnp.float32),
            pltpu.VMEM((16,), jnp.float32),
            pltpu.VMEM((16,), jnp.float32),
            pltpu.VMEM((128,), jnp.int32),
            pltpu.VMEM((128,), jnp.int32),
            pltpu.SemaphoreType.DMA,
            pltpu.SemaphoreType.DMA,
            pltpu.SemaphoreType.DMA,
            pltpu.SemaphoreType.DMA,
            pltpu.VMEM_SHARED((256, 16), jnp.float32),
            pltpu.VMEM_SHARED((256, 16), jnp.float32),
        ],
    )
    partials, _neg = fn(y2, o2, w2, ts_pad)
    train_loss = (partials[0, 0] + partials[1, 0]) / _B
    mse_mean = (partials[0, 1] + partials[1, 1]) / (_B * _N)
    return ((train_loss + mse_mean) * 10).reshape(())


# speedup vs baseline: 9.4512x; 2.6926x over previous
import functools

import jax
import jax.numpy as jnp
from jax import lax
from jax.experimental import pallas as pl
from jax.experimental.pallas import tpu as pltpu
from jax.experimental.pallas import tpu_sc as plsc

_NUM_CLASSES = 8
_NEG_POS_RATIO = 3

_B, _H, _W, _C = 4, 512, 512, _NUM_CLASSES
_N = _H * _W * _C
_NCORE = 2
_NSUB = 16
_BPC = _B // _NCORE
_NSL = _N // _NSUB
_CH = 8192
_NCHUNK = _NSL // _CH
_VPC = _CH // 16
_UNROLL = 4
_POS_ROW = 128
_MSE_ROW = 129


def _sc_body(y_hbm, o_hbm, w_hbm, ts_hbm, out_hbm, neg_hbm,
             ybuf, obuf, wbuf, negbuf, ybuf2, obuf2, wbuf2, negbuf2,
             cnt_h, sum_h, gcnt, gsum, tsbuf, outbuf, idx_lo, idx_hi,
             sem_a, sem_b, sem_na, sem_nb, sh_cnt, sh_sum):
    core = lax.axis_index("c")
    sid = lax.axis_index("s")
    lanes = lax.iota(jnp.int32, 16)
    onesf = jnp.full((16,), 1.0, jnp.float32)
    zerof = jnp.zeros((16,), jnp.float32)

    def init_idx(i, _):
        v = lanes + i * 16
        idx_lo[pl.ds(i * 16, 16)] = v
        idx_hi[pl.ds(i * 16, 16)] = v + 128
        return 0

    lax.fori_loop(0, 8, init_idx, 0)
    pltpu.sync_copy(ts_hbm, tsbuf)

    def reset_hists():
        plsc.subcore_barrier()

        @plsc.parallel_loop(0, _UNROLL * 256, unroll=8)
        def _(i):
            cnt_h[i] = zerof
            sum_h[i] = zerof

        @pl.when(sid == 0)
        def _():
            pltpu.sync_copy(cnt_h.at[pl.ds(0, 256)], sh_cnt)
            pltpu.sync_copy(sum_h.at[pl.ds(0, 256)], sh_sum)

        plsc.subcore_barrier()

    def combine_hists():
        for sec in range(_UNROLL):
            r = sec * 256
            pltpu.sync_copy(cnt_h.at[pl.ds(r, 128)], sh_cnt.at[idx_lo],
                            add=True)
            pltpu.sync_copy(cnt_h.at[pl.ds(r + 128, 128)], sh_cnt.at[idx_hi],
                            add=True)
            pltpu.sync_copy(sum_h.at[pl.ds(r, 128)], sh_sum.at[idx_lo],
                            add=True)
            pltpu.sync_copy(sum_h.at[pl.ds(r + 128, 128)], sh_sum.at[idx_hi],
                            add=True)
        plsc.subcore_barrier()
        pltpu.sync_copy(sh_cnt, gcnt)
        pltpu.sync_copy(sh_sum, gsum)

    def search(kk, top_bucket):
        def body(j, carry):
            cum, above, t, r, cnt_t, sum_t, found = carry
            b = top_bucket - 1 - j
            cj = jnp.sum(gcnt[b])
            sj = jnp.sum(gsum[b])
            here = jnp.logical_and(jnp.logical_not(found), cum + cj >= kk)
            t = jnp.where(here, b, t)
            r = jnp.where(here, kk - cum, r)
            cnt_t = jnp.where(here, cj, cnt_t)
            sum_t = jnp.where(here, sj, sum_t)
            above = jnp.where(jnp.logical_or(found, here), above, above + sj)
            return (cum + cj, above, t, r, cnt_t, sum_t,
                    jnp.logical_or(found, here))

        init = (jnp.float32(0), jnp.float32(0), jnp.int32(-1), jnp.float32(0),
                jnp.float32(0), jnp.float32(0), False)
        _, above, t, r, cnt_t, sum_t, _ = lax.fori_loop(
            0, top_bucket, body, init)
        return above, t, r, cnt_t, sum_t

    def start_loads(off, yb, ob, wb, sem):
        pltpu.async_copy(y_hbm.at[pl.ds(off, _CH)], yb, sem)
        pltpu.async_copy(o_hbm.at[pl.ds(off, _CH)], ob, sem)
        pltpu.async_copy(w_hbm.at[pl.ds(off, _CH)], wb, sem)

    def wait_loads(yb, ob, wb, sem):
        pltpu.make_async_copy(y_hbm.at[pl.ds(0, _CH)], yb, sem).wait()
        pltpu.make_async_copy(o_hbm.at[pl.ds(0, _CH)], ob, sem).wait()
        pltpu.make_async_copy(w_hbm.at[pl.ds(0, _CH)], wb, sem).wait()

    def start_neg_load(off, nb, sem):
        pltpu.async_copy(neg_hbm.at[pl.ds(off, _CH)], nb, sem)

    def wait_neg_load(nb, sem):
        pltpu.make_async_copy(neg_hbm.at[pl.ds(0, _CH)], nb, sem).wait()

    def start_neg_store(nb, off, sem):
        pltpu.async_copy(nb, neg_hbm.at[pl.ds(off, _CH)], sem)

    def wait_neg_store(nb, sem):
        pltpu.make_async_copy(nb, neg_hbm.at[pl.ds(0, _CH)], sem).wait()

    def compute_chunk(yb, ob, wb, nb, carry):
        @plsc.parallel_loop(0, _VPC, unroll=_UNROLL, carry=carry)
        def vbody(vi, c2):
            pa, ma = c2
            sec = (vi & (_UNROLL - 1)) << 8
            sl = pl.ds(vi * 16, 16)
            yv = yb[sl]
            ov = ob[sl]
            wv = wb[sl]
            d = ov - yv
            m = d * d
            ma = ma + m
            posm = wv > 0
            pa = pa + jnp.where(posm, wv * m, 0.0)
            negm = jnp.logical_and(ov > 0, jnp.logical_not(posm))
            nv = jnp.where(negm, m, 0.0)
            nb[sl] = nv
            bits = plsc.bitcast(nv, jnp.int32)
            b1 = (bits >> 24) + sec
            plsc.addupdate_scatter(cnt_h, [b1, lanes], onesf)
            plsc.addupdate_scatter(sum_h, [b1, lanes], nv)
            return (pa, ma)

        return vbody

    def sweep1(base):
        start_loads(base, ybuf, obuf, wbuf, sem_a)

        def pair(i, carry):
            off_a = base + (2 * i) * _CH
            wait_loads(ybuf, obuf, wbuf, sem_a)
            start_loads(off_a + _CH, ybuf2, obuf2, wbuf2, sem_b)

            @pl.when(i > 0)
            def _():
                wait_neg_store(negbuf, sem_na)

            carry = compute_chunk(ybuf, obuf, wbuf, negbuf, carry)
            start_neg_store(negbuf, off_a, sem_na)

            wait_loads(ybuf2, obuf2, wbuf2, sem_b)

            @pl.when(i < _NCHUNK // 2 - 1)
            def _():
                start_loads(off_a + 2 * _CH, ybuf, obuf, wbuf, sem_a)

            @pl.when(i > 0)
            def _():
                wait_neg_store(negbuf2, sem_nb)

            carry = compute_chunk(ybuf2, obuf2, wbuf2, negbuf2, carry)
            start_neg_store(negbuf2, off_a + _CH, sem_nb)
            return carry

        pos_a, mse_a = lax.fori_loop(0, _NCHUNK // 2, pair, (zerof, zerof))
        wait_neg_store(negbuf, sem_na)
        wait_neg_store(negbuf2, sem_nb)
        cnt_h[_POS_ROW] = pos_a
        cnt_h[_MSE_ROW] = mse_a

    def hist_chunk(nb, shift, prev_shift, prefix):
        @plsc.parallel_loop(0, _VPC, unroll=_UNROLL)
        def vbody(vi):
            sec = (vi & (_UNROLL - 1)) << 8
            sl = pl.ds(vi * 16, 16)
            nv = nb[sl]
            bits = plsc.bitcast(nv, jnp.int32)
            sel = (bits >> prev_shift) == prefix
            bb = ((bits >> shift) & 0xFF) + sec
            plsc.addupdate_scatter(cnt_h, [bb, lanes], onesf, mask=sel)
            plsc.addupdate_scatter(sum_h, [bb, lanes], nv, mask=sel)

    def sweep_refine(base, shift, prev_shift, prefix):
        start_neg_load(base, ybuf, sem_a)

        def pair(i, _):
            off_a = base + (2 * i) * _CH
            wait_neg_load(ybuf, sem_a)
            start_neg_load(off_a + _CH, ybuf2, sem_b)
            hist_chunk(ybuf, shift, prev_shift, prefix)
            wait_neg_load(ybuf2, sem_b)

            @pl.when(i < _NCHUNK // 2 - 1)
            def _():
                start_neg_load(off_a + 2 * _CH, ybuf, sem_a)

            hist_chunk(ybuf2, shift, prev_shift, prefix)
            return 0

        lax.fori_loop(0, _NCHUNK // 2, pair, 0)

    kvec = jnp.minimum(tsbuf[...].astype(jnp.int32) * _NEG_POS_RATIO,
                       _N).astype(jnp.float32)

    def batch_body(bi, acc):
        loss_acc, mse_acc = acc
        b = core * _BPC + bi
        base = b * _N + sid * _NSL
        bsel = lanes == b
        kk = jnp.sum(jnp.where(bsel, kvec, 0.0))
        tsb = jnp.sum(jnp.where(bsel, tsbuf[...], 0.0))

        with jax.named_scope("reset1"):
            reset_hists()
        with jax.named_scope("sweep1"):
            sweep1(base)
        with jax.named_scope("combine1"):
            combine_hists()
        with jax.named_scope("search1"):
            above1, t1, r1, _, _ = search(kk, 128)
        pos_b = jnp.sum(gcnt[_POS_ROW])
        mse_b = jnp.sum(gcnt[_MSE_ROW])

        with jax.named_scope("reset2"):
            reset_hists()
        with jax.named_scope("sweep2"):
            sweep_refine(base, 16, 24, t1)
        with jax.named_scope("combine2"):
            combine_hists()
        with jax.named_scope("search2"):
            above2, t2, r2, _, _ = search(r1, 256)

        with jax.named_scope("reset3"):
            reset_hists()
        with jax.named_scope("sweep3"):
            sweep_refine(base, 8, 16, (t1 << 8) | t2)
        with jax.named_scope("combine3"):
            combine_hists()
        with jax.named_scope("search3"):
            above3, _, r3, cnt3, sum3 = search(r2, 256)

        mean3_v = jnp.full((16,), sum3) / jnp.maximum(jnp.full((16,), cnt3),
                                                      1.0)
        num_v = jnp.full((16,), pos_b + above1 + above2 + above3) \
            + r3 * mean3_v
        ts_v = jnp.full((16,), tsb)
        safe_ts = jnp.where(ts_v > 0, ts_v, 1.0)
        loss_acc = loss_acc + jnp.where(ts_v > 0, num_v / safe_ts, 0.0)
        mse_acc = mse_acc + mse_b
        return (loss_acc, mse_acc)

    loss_acc, mse_acc = lax.fori_loop(0, _BPC, batch_body, (zerof, zerof))

    plsc.subcore_barrier()

    @pl.when(sid == 0)
    def _():
        outv = jnp.where(lanes == 0, loss_acc,
                         jnp.where(lanes == 1, mse_acc, 0.0))
        outbuf[...] = outv
        pltpu.sync_copy(outbuf, out_hbm.at[core])


@functools.partial(jax.jit, static_argnames=())
def kernel(y, out, w, total_size):
    assert y.shape == (_B, _H, _W, _C)

    def _flat_physical(a):
        return a.reshape(_B, _H, _W // 128, 128, _C).transpose(
            0, 1, 2, 4, 3).reshape(-1)

    y2 = _flat_physical(y)
    o2 = _flat_physical(out)
    w2 = _flat_physical(w)
    ts_pad = jnp.zeros((16,), jnp.float32).at[:_B].set(
        total_size.reshape(-1).astype(jnp.float32))

    mesh = plsc.VectorSubcoreMesh(core_axis_name="c", subcore_axis_name="s",
                                  num_cores=_NCORE, num_subcores=_NSUB)
    fn = pl.kernel(
        _sc_body,
        out_type=(
            jax.ShapeDtypeStruct((_NCORE, 16), jnp.float32),
            jax.ShapeDtypeStruct((_B * _N,), jnp.float32),
        ),
        mesh=mesh,
        compiler_params=pltpu.CompilerParams(needs_layout_passes=False,
                                             use_tc_tiling_on_sc=False),
        scratch_types=[
            pltpu.VMEM((_CH,), jnp.float32),
            pltpu.VMEM((_CH,), jnp.float32),
            pltpu.VMEM((_CH,), jnp.float32),
            pltpu.VMEM((_CH,), jnp.float32),
            pltpu.VMEM((_CH,), jnp.float32),
            pltpu.VMEM((_CH,), jnp.float32),
            pltpu.VMEM((_CH,), jnp.float32),
            pltpu.VMEM((_CH,), jnp.float32),
            pltpu.VMEM((_UNROLL * 256, 16), jnp.float32),
            pltpu.VMEM((_UNROLL * 256, 16), jnp.float32),
            pltpu.VMEM((256, 16), jnp.float32),
            pltpu.VMEM((256, 16), jnp.float32),
            pltpu.VMEM((16,), jnp.float32),
            pltpu.VMEM((16,), jnp.float32),
            pltpu.VMEM((128,), jnp.int32),
            pltpu.VMEM((128,), jnp.int32),
            pltpu.SemaphoreType.DMA,
            pltpu.SemaphoreType.DMA,
            pltpu.SemaphoreType.DMA,
            pltpu.SemaphoreType.DMA,
            pltpu.VMEM_SHARED((256, 16), jnp.float32),
            pltpu.VMEM_SHARED((256, 16), jnp.float32),
        ],
    )
    partials, _neg = fn(y2, o2, w2, ts_pad)
    train_loss = (partials[0, 0] + partials[1, 0]) / _B
    mse_mean = (partials[0, 1] + partials[1, 1]) / (_B * _N)
    return ((train_loss + mse_mean) * 10).reshape(())
